# baseline (speedup 1.0000x reference)
import jax
import jax.numpy as jnp
from jax import lax
from jax.experimental import pallas as pl
from jax.experimental.pallas import tpu as pltpu
from jax.experimental.pallas import tpu_sc as plsc

K = 32
BATCH = 16384
NC = 2
NS = 16
NW = NC * NS
BPW = BATCH // NW
WPW = BPW * K
L = 16
HCH = 256
NH = BPW // HCH
GPH = HCH // L


def _sc_body(uwidx_hbm, vwidx_hbm, wflat_hbm, hflat_hbm, dot_hbm, emb_hbm,
             idx_u, idx_v, u_flat, v_flat, cat, dvec, sem, sem_out):
    wid = lax.axis_index("s") * NC + lax.axis_index("c")
    base = wid * BPW

    pltpu.sync_copy(uwidx_hbm.at[wid], idx_u)
    pltpu.sync_copy(vwidx_hbm.at[wid], idx_v)

    du = pltpu.async_copy(wflat_hbm.at[idx_u], u_flat, sem)
    dv = pltpu.async_copy(hflat_hbm.at[idx_v], v_flat, sem)
    du.wait()
    dv.wait()

    iota16 = lax.iota(jnp.int32, L)

    for h in range(NH):
        def grp(g, carry, h=h):
            lrows = g * L + iota16
            fbase = (h * HCH + g * L) * K
            acc = jnp.zeros((L,), jnp.float32)
            for k in range(K):
                fr = fbase + k + iota16 * K
                kv = jnp.full((L,), k, jnp.int32)
                kv2 = jnp.full((L,), K + k, jnp.int32)
                u = plsc.load_gather(u_flat, [fr])
                v = plsc.load_gather(v_flat, [fr])
                plsc.store_scatter(cat, [lrows, kv], u)
                plsc.store_scatter(cat, [lrows, kv2], v)
                acc = acc + u * v
            dvec[pl.ds(h * HCH + g * L, L)] = acc
            return carry

        lax.fori_loop(0, GPH, grp, 0)
        pltpu.async_copy(
            cat, emb_hbm.at[pl.ds(base + h * HCH, HCH)], sem_out).wait()

    pltpu.sync_copy(dvec, dot_hbm.at[pl.ds(base, BPW)])


@jax.jit
def _mf_dr(uwidx, vwidx, wflat, hflat):
    mesh = plsc.VectorSubcoreMesh(core_axis_name="c", subcore_axis_name="s")
    return pl.kernel(
        _sc_body,
        out_type=(
            jax.ShapeDtypeStruct((BATCH,), jnp.float32),
            jax.ShapeDtypeStruct((BATCH, 2 * K), jnp.float32),
        ),
        mesh=mesh,
        compiler_params=pltpu.CompilerParams(
            use_tc_tiling_on_sc=False, needs_layout_passes=False),
        scratch_types=[
            pltpu.VMEM((WPW,), jnp.int32),
            pltpu.VMEM((WPW,), jnp.int32),
            pltpu.VMEM((WPW,), jnp.float32),
            pltpu.VMEM((WPW,), jnp.float32),
            pltpu.VMEM((HCH, 2 * K), jnp.float32),
            pltpu.VMEM((BPW,), jnp.float32),
            pltpu.SemaphoreType.DMA,
            pltpu.SemaphoreType.DMA,
        ],
        name="mf_dr_sc",
    )(uwidx, vwidx, wflat, hflat)


def kernel(x, W, H):
    ar = jnp.arange(K, dtype=jnp.int32)
    uwidx = (x[:, 0:1].astype(jnp.int32) * K + ar).reshape(NW, WPW)
    vwidx = (x[:, 1:2].astype(jnp.int32) * K + ar).reshape(NW, WPW)
    out, emb = _mf_dr(uwidx, vwidx, W.reshape(-1), H.reshape(-1))
    return (out, emb)
